# baseline (speedup 1.0000x reference)
import jax
import jax.numpy as jnp
import numpy as np
from jax import lax
from jax.experimental import pallas as pl
from jax.experimental.pallas import tpu as pltpu
from jax.experimental.pallas import tpu_sc as plsc

GRID_SCALE = 0.0001

TRI_TABLE = np.array([
    [-1, -1, -1, -1, -1, -1],
    [1, 0, 2, -1, -1, -1],
    [4, 0, 3, -1, -1, -1],
    [1, 4, 2, 1, 3, 4],
    [3, 1, 5, -1, -1, -1],
    [2, 3, 0, 2, 5, 3],
    [1, 4, 0, 1, 5, 4],
    [4, 2, 5, -1, -1, -1],
    [4, 5, 2, -1, -1, -1],
    [4, 1, 0, 4, 5, 1],
    [3, 2, 0, 3, 5, 2],
    [1, 3, 5, -1, -1, -1],
    [4, 1, 2, 4, 3, 1],
    [3, 0, 4, -1, -1, -1],
    [2, 0, 1, -1, -1, -1],
    [-1, -1, -1, -1, -1, -1]], dtype=np.int32)

EDGE_A = (0, 0, 0, 1, 1, 2)
EDGE_B = (1, 2, 3, 2, 3, 3)

NC = 2
NS = 16
L = 16
NW = NC * NS

F = 300000
N = 100000

C = 640
G = C // L
CH = 15
TPW = C * CH
F_PAD = TPW * NW


def _pack_body(tv_ref, df_ref, ox_ref, oy_ref, oz_ref):
    v = tv_ref[...] + jnp.tanh(df_ref[...]) * (GRID_SCALE / 2.0)
    ox_ref[...] = v[:, 0:1]
    oy_ref[...] = v[:, 1:2]
    oz_ref[...] = v[:, 2:3]


def _deform_verts(tet_v, deform):
    bn = 2000
    shp = jax.ShapeDtypeStruct((N, 1), jnp.float32)
    spec = pl.BlockSpec((bn, 1), lambda i: (i, 0))
    ox, oy, oz = pl.pallas_call(
        _pack_body,
        grid=(N // bn,),
        in_specs=[
            pl.BlockSpec((bn, 3), lambda i: (i, 0)),
            pl.BlockSpec((bn, 3), lambda i: (i, 0)),
        ],
        out_specs=[spec, spec, spec],
        out_shape=[shp, shp, shp],
    )(tet_v, deform)
    return ox.reshape(N), oy.reshape(N), oz.reshape(N)


def _mt_body(x_hbm, y_hbm, z_hbm, s_hbm, ti_hbm, tt_hbm,
             mv_hbm, fc_hbm, vl_hbm,
             idx_v, gx_v, gy_v, gz_v, gs_v,
             tt_v, mv_v, fc_v, vl_v, sem):
    wid = lax.axis_index("s") * NC + lax.axis_index("c")
    wbase = wid * TPW

    pltpu.sync_copy(tt_hbm, tt_v)

    lane = lax.broadcasted_iota(jnp.int32, (L,), 0)

    def chunk_body(g, carry):
        t0 = wbase + g * C
        pltpu.sync_copy(ti_hbm.at[pl.ds(t0 * 4, 4 * C)], idx_v)
        copies = [pltpu.async_copy(x_hbm.at[idx_v], gx_v, sem),
                  pltpu.async_copy(y_hbm.at[idx_v], gy_v, sem),
                  pltpu.async_copy(z_hbm.at[idx_v], gz_v, sem),
                  pltpu.async_copy(s_hbm.at[idx_v], gs_v, sem)]
        for cp in copies:
            cp.wait()

        def group_body(j, c2):
            tloc = j * L + lane
            rbase = tloc * 4
            X = [plsc.load_gather(gx_v, [rbase + a]) for a in range(4)]
            Y = [plsc.load_gather(gy_v, [rbase + a]) for a in range(4)]
            Z = [plsc.load_gather(gz_v, [rbase + a]) for a in range(4)]
            S = [plsc.load_gather(gs_v, [rbase + a]) for a in range(4)]

            code = (S[0] > 0.0).astype(jnp.int32)
            code = code + 2 * (S[1] > 0.0).astype(jnp.int32)
            code = code + 4 * (S[2] > 0.0).astype(jnp.int32)
            code = code + 8 * (S[3] > 0.0).astype(jnp.int32)
            code6 = code * 6

            gt6 = (t0 + tloc) * 6
            f6 = tloc * 6
            valid = []
            for col in range(6):
                ent = plsc.load_gather(tt_v, [code6 + col])
                v_ok = ent >= 0
                valid.append(v_ok)
                fval = gt6 + jnp.where(v_ok, ent, 0)
                plsc.store_scatter(fc_v, [f6 + col], fval)
            for r in range(2):
                v_ok = valid[3 * r] & valid[3 * r + 1] & valid[3 * r + 2]
                plsc.store_scatter(vl_v, [2 * tloc + r],
                                   v_ok.astype(jnp.int32))

            m18 = tloc * 18
            for e in range(6):
                a, b = EDGE_A[e], EDGE_B[e]
                d = S[b] - S[a]
                small = jnp.abs(d) < 1e-10
                w = jnp.where(small, 0.5,
                              S[b] / jnp.where(small, 1.0, d))
                u = 1.0 - w
                base = m18 + e * 3
                plsc.store_scatter(mv_v, [base], X[a] * w + X[b] * u)
                plsc.store_scatter(mv_v, [base + 1], Y[a] * w + Y[b] * u)
                plsc.store_scatter(mv_v, [base + 2], Z[a] * w + Z[b] * u)
            return c2

        lax.fori_loop(0, G, group_body, 0)

        pltpu.sync_copy(mv_v, mv_hbm.at[pl.ds(t0 * 18, 18 * C)])
        pltpu.sync_copy(fc_v, fc_hbm.at[pl.ds(t0 * 6, 6 * C)])
        pltpu.sync_copy(vl_v, vl_hbm.at[pl.ds(t0 * 2, 2 * C)])
        return carry

    lax.fori_loop(0, CH, chunk_body, 0)


def _marching(x, y, z, s, ti_flat, tt):
    mesh = plsc.VectorSubcoreMesh(core_axis_name="c", subcore_axis_name="s")
    val_t = pltpu.VMEM((4 * C,), jnp.float32)
    fn = pl.kernel(
        _mt_body, mesh=mesh,
        out_type=[
            jax.ShapeDtypeStruct((F_PAD * 18,), jnp.float32),
            jax.ShapeDtypeStruct((F_PAD * 6,), jnp.int32),
            jax.ShapeDtypeStruct((F_PAD * 2,), jnp.int32),
        ],
        scratch_types=[
            pltpu.VMEM((4 * C,), jnp.int32),
            val_t, val_t, val_t, val_t,
            pltpu.VMEM((96,), jnp.int32),
            pltpu.VMEM((18 * C,), jnp.float32),
            pltpu.VMEM((6 * C,), jnp.int32),
            pltpu.VMEM((2 * C,), jnp.int32),
            pltpu.SemaphoreType.DMA,
        ],
        compiler_params=pltpu.CompilerParams(needs_layout_passes=False),
    )
    return fn(x, y, z, s, ti_flat, tt)


def kernel(tet_v, sdf, deform, tet_ind):
    x, y, z = _deform_verts(tet_v, deform)
    ti_flat = jnp.pad(tet_ind, ((0, F_PAD - F), (0, 0))).reshape(F_PAD * 4)
    tt = jnp.asarray(TRI_TABLE).reshape(96)
    mv, fc, vl = _marching(x, y, z, sdf, ti_flat, tt)
    mesh_verts = mv.reshape(F_PAD * 6, 3)[:F * 6]
    faces = fc.reshape(F_PAD * 2, 3)[:F * 2]
    face_valid = vl[:F * 2].astype(bool)
    return mesh_verts, faces, face_valid


# speedup vs baseline: 1.2564x; 1.2564x over previous
import jax
import jax.numpy as jnp
import numpy as np
from jax import lax
from jax.experimental import pallas as pl
from jax.experimental.pallas import tpu as pltpu
from jax.experimental.pallas import tpu_sc as plsc

GRID_SCALE = 0.0001

TRI_TABLE = np.array([
    [-1, -1, -1, -1, -1, -1],
    [1, 0, 2, -1, -1, -1],
    [4, 0, 3, -1, -1, -1],
    [1, 4, 2, 1, 3, 4],
    [3, 1, 5, -1, -1, -1],
    [2, 3, 0, 2, 5, 3],
    [1, 4, 0, 1, 5, 4],
    [4, 2, 5, -1, -1, -1],
    [4, 5, 2, -1, -1, -1],
    [4, 1, 0, 4, 5, 1],
    [3, 2, 0, 3, 5, 2],
    [1, 3, 5, -1, -1, -1],
    [4, 1, 2, 4, 3, 1],
    [3, 0, 4, -1, -1, -1],
    [2, 0, 1, -1, -1, -1],
    [-1, -1, -1, -1, -1, -1]], dtype=np.int32)

EDGE_A = (0, 0, 0, 1, 1, 2)
EDGE_B = (1, 2, 3, 2, 3, 3)

NC = 2
NS = 16
L = 16
NW = NC * NS

F = 300000
N = 100000

C = 640
G = C // L
CH = 15
TPW = C * CH
F_PAD = TPW * NW


def _pack_body(tv_ref, df_ref, ox_ref, oy_ref, oz_ref):
    v = tv_ref[...] + jnp.tanh(df_ref[...]) * (GRID_SCALE / 2.0)
    ox_ref[...] = v[:, 0:1]
    oy_ref[...] = v[:, 1:2]
    oz_ref[...] = v[:, 2:3]


def _deform_verts(tet_v, deform):
    bn = 2000
    shp = jax.ShapeDtypeStruct((N, 1), jnp.float32)
    spec = pl.BlockSpec((bn, 1), lambda i: (i, 0))
    ox, oy, oz = pl.pallas_call(
        _pack_body,
        grid=(N // bn,),
        in_specs=[
            pl.BlockSpec((bn, 3), lambda i: (i, 0)),
            pl.BlockSpec((bn, 3), lambda i: (i, 0)),
        ],
        out_specs=[spec, spec, spec],
        out_shape=[shp, shp, shp],
    )(tet_v, deform)
    return ox.reshape(N), oy.reshape(N), oz.reshape(N)


def _mt_body(x_hbm, y_hbm, z_hbm, s_hbm, ti_hbm, tt_hbm,
             mv_hbm, fc_hbm, vl_hbm,
             xs_sh, ys_sh, zs_sh, ss_sh,
             idx_v, gx_v, gy_v, gz_v, gs_v,
             tt_v, mv_v, fc_v, vl_v, sem):
    cid = lax.axis_index("c")
    sid = lax.axis_index("s")
    wid = sid * NC + cid
    wbase = wid * TPW

    @pl.when(sid == 0)
    def _():
        pltpu.sync_copy(x_hbm, xs_sh)

    @pl.when(sid == 1)
    def _():
        pltpu.sync_copy(y_hbm, ys_sh)

    @pl.when(sid == 2)
    def _():
        pltpu.sync_copy(z_hbm, zs_sh)

    @pl.when(sid == 3)
    def _():
        pltpu.sync_copy(s_hbm, ss_sh)

    pltpu.sync_copy(tt_hbm, tt_v)
    plsc.subcore_barrier()

    lane = lax.broadcasted_iota(jnp.int32, (L,), 0)

    def chunk_body(g, carry):
        t0 = wbase + g * C
        pltpu.sync_copy(ti_hbm.at[pl.ds(t0 * 4, 4 * C)], idx_v)
        copies = [pltpu.async_copy(xs_sh.at[idx_v], gx_v, sem),
                  pltpu.async_copy(ys_sh.at[idx_v], gy_v, sem),
                  pltpu.async_copy(zs_sh.at[idx_v], gz_v, sem),
                  pltpu.async_copy(ss_sh.at[idx_v], gs_v, sem)]
        for cp in copies:
            cp.wait()

        def group_body(j, c2):
            tloc = j * L + lane
            rbase = tloc * 4
            X = [plsc.load_gather(gx_v, [rbase + a]) for a in range(4)]
            Y = [plsc.load_gather(gy_v, [rbase + a]) for a in range(4)]
            Z = [plsc.load_gather(gz_v, [rbase + a]) for a in range(4)]
            S = [plsc.load_gather(gs_v, [rbase + a]) for a in range(4)]

            code = (S[0] > 0.0).astype(jnp.int32)
            code = code + 2 * (S[1] > 0.0).astype(jnp.int32)
            code = code + 4 * (S[2] > 0.0).astype(jnp.int32)
            code = code + 8 * (S[3] > 0.0).astype(jnp.int32)
            code6 = code * 6

            gt6 = (t0 + tloc) * 6
            f6 = tloc * 6
            valid = []
            for col in range(6):
                ent = plsc.load_gather(tt_v, [code6 + col])
                v_ok = ent >= 0
                valid.append(v_ok)
                fval = gt6 + jnp.where(v_ok, ent, 0)
                plsc.store_scatter(fc_v, [f6 + col], fval)
            for r in range(2):
                v_ok = valid[3 * r] & valid[3 * r + 1] & valid[3 * r + 2]
                plsc.store_scatter(vl_v, [2 * tloc + r],
                                   v_ok.astype(jnp.int32))

            m18 = tloc * 18
            for e in range(6):
                a, b = EDGE_A[e], EDGE_B[e]
                d = S[b] - S[a]
                small = jnp.abs(d) < 1e-10
                w = jnp.where(small, 0.5,
                              S[b] / jnp.where(small, 1.0, d))
                u = 1.0 - w
                base = m18 + e * 3
                plsc.store_scatter(mv_v, [base], X[a] * w + X[b] * u)
                plsc.store_scatter(mv_v, [base + 1], Y[a] * w + Y[b] * u)
                plsc.store_scatter(mv_v, [base + 2], Z[a] * w + Z[b] * u)
            return c2

        lax.fori_loop(0, G, group_body, 0)

        pltpu.sync_copy(mv_v, mv_hbm.at[pl.ds(t0 * 18, 18 * C)])
        pltpu.sync_copy(fc_v, fc_hbm.at[pl.ds(t0 * 6, 6 * C)])
        pltpu.sync_copy(vl_v, vl_hbm.at[pl.ds(t0 * 2, 2 * C)])
        return carry

    lax.fori_loop(0, CH, chunk_body, 0)


def _marching(x, y, z, s, ti_flat, tt):
    mesh = plsc.VectorSubcoreMesh(core_axis_name="c", subcore_axis_name="s")
    val_t = pltpu.VMEM((4 * C,), jnp.float32)
    fn = pl.kernel(
        _mt_body, mesh=mesh,
        out_type=[
            jax.ShapeDtypeStruct((F_PAD * 18,), jnp.float32),
            jax.ShapeDtypeStruct((F_PAD * 6,), jnp.int32),
            jax.ShapeDtypeStruct((F_PAD * 2,), jnp.int32),
        ],
        scratch_types=[
            pltpu.VMEM_SHARED((N,), jnp.float32),
            pltpu.VMEM_SHARED((N,), jnp.float32),
            pltpu.VMEM_SHARED((N,), jnp.float32),
            pltpu.VMEM_SHARED((N,), jnp.float32),
            pltpu.VMEM((4 * C,), jnp.int32),
            val_t, val_t, val_t, val_t,
            pltpu.VMEM((96,), jnp.int32),
            pltpu.VMEM((18 * C,), jnp.float32),
            pltpu.VMEM((6 * C,), jnp.int32),
            pltpu.VMEM((2 * C,), jnp.int32),
            pltpu.SemaphoreType.DMA,
        ],
        compiler_params=pltpu.CompilerParams(needs_layout_passes=False),
    )
    return fn(x, y, z, s, ti_flat, tt)


def kernel(tet_v, sdf, deform, tet_ind):
    x, y, z = _deform_verts(tet_v, deform)
    ti_flat = jnp.pad(tet_ind, ((0, F_PAD - F), (0, 0))).reshape(F_PAD * 4)
    tt = jnp.asarray(TRI_TABLE).reshape(96)
    mv, fc, vl = _marching(x, y, z, sdf, ti_flat, tt)
    mesh_verts = mv.reshape(F_PAD * 6, 3)[:F * 6]
    faces = fc.reshape(F_PAD * 2, 3)[:F * 2]
    face_valid = vl[:F * 2].astype(bool)
    return mesh_verts, faces, face_valid


# speedup vs baseline: 1.4061x; 1.1191x over previous
import jax
import jax.numpy as jnp
import numpy as np
from jax import lax
from jax.experimental import pallas as pl
from jax.experimental.pallas import tpu as pltpu
from jax.experimental.pallas import tpu_sc as plsc

GRID_SCALE = 0.0001

TRI_TABLE = np.array([
    [-1, -1, -1, -1, -1, -1],
    [1, 0, 2, -1, -1, -1],
    [4, 0, 3, -1, -1, -1],
    [1, 4, 2, 1, 3, 4],
    [3, 1, 5, -1, -1, -1],
    [2, 3, 0, 2, 5, 3],
    [1, 4, 0, 1, 5, 4],
    [4, 2, 5, -1, -1, -1],
    [4, 5, 2, -1, -1, -1],
    [4, 1, 0, 4, 5, 1],
    [3, 2, 0, 3, 5, 2],
    [1, 3, 5, -1, -1, -1],
    [4, 1, 2, 4, 3, 1],
    [3, 0, 4, -1, -1, -1],
    [2, 0, 1, -1, -1, -1],
    [-1, -1, -1, -1, -1, -1]], dtype=np.int32)

EDGE_A = (0, 0, 0, 1, 1, 2)
EDGE_B = (1, 2, 3, 2, 3, 3)

NC = 2
NS = 16
L = 16
NW = NC * NS

F = 300000
N = 100000

C = 640
G = C // L
CH = 15
NGRP = F // L
BASE_G = NGRP // NW
REM_G = NGRP % NW


def _pack_body(tv_ref, df_ref, ox_ref, oy_ref, oz_ref):
    v = tv_ref[...] + jnp.tanh(df_ref[...]) * (GRID_SCALE / 2.0)
    ox_ref[...] = v[:, 0]
    oy_ref[...] = v[:, 1]
    oz_ref[...] = v[:, 2]


def _deform_verts(tet_v, deform):
    bn = 2048
    shp = jax.ShapeDtypeStruct((N,), jnp.float32)
    spec = pl.BlockSpec((bn,), lambda i: (i,))
    return pl.pallas_call(
        _pack_body,
        grid=((N + bn - 1) // bn,),
        in_specs=[
            pl.BlockSpec((bn, 3), lambda i: (i, 0)),
            pl.BlockSpec((bn, 3), lambda i: (i, 0)),
        ],
        out_specs=[spec, spec, spec],
        out_shape=[shp, shp, shp],
    )(tet_v, deform)


def _mt_body(x_hbm, y_hbm, z_hbm, s_hbm, ti_hbm, tt_hbm,
             mv_hbm, fc_hbm, vl_hbm,
             xs_sh, ys_sh, zs_sh, ss_sh,
             idx_v, gx_v, gy_v, gz_v, gs_v,
             tt_v, mv_v, fc_v, vl_v, sem):
    cid = lax.axis_index("c")
    sid = lax.axis_index("s")
    wid = sid * NC + cid
    t0w = (wid * BASE_G + jnp.minimum(wid, REM_G)) * L
    tend = t0w + (BASE_G + jnp.where(wid < REM_G, 1, 0)) * L

    @pl.when(sid == 0)
    def _():
        pltpu.sync_copy(x_hbm, xs_sh)

    @pl.when(sid == 1)
    def _():
        pltpu.sync_copy(y_hbm, ys_sh)

    @pl.when(sid == 2)
    def _():
        pltpu.sync_copy(z_hbm, zs_sh)

    @pl.when(sid == 3)
    def _():
        pltpu.sync_copy(s_hbm, ss_sh)

    pltpu.sync_copy(tt_hbm, tt_v)
    plsc.subcore_barrier()

    lane = lax.broadcasted_iota(jnp.int32, (L,), 0)

    def chunk_body(g, carry):
        t0 = jnp.minimum(t0w + g * C, tend - C)
        pltpu.sync_copy(ti_hbm.at[pl.ds(t0 * 4, 4 * C)], idx_v)
        copies = [pltpu.async_copy(xs_sh.at[idx_v], gx_v, sem),
                  pltpu.async_copy(ys_sh.at[idx_v], gy_v, sem),
                  pltpu.async_copy(zs_sh.at[idx_v], gz_v, sem),
                  pltpu.async_copy(ss_sh.at[idx_v], gs_v, sem)]
        for cp in copies:
            cp.wait()

        def group_body(j, c2):
            tloc = j * L + lane
            rbase = tloc * 4
            X = [plsc.load_gather(gx_v, [rbase + a]) for a in range(4)]
            Y = [plsc.load_gather(gy_v, [rbase + a]) for a in range(4)]
            Z = [plsc.load_gather(gz_v, [rbase + a]) for a in range(4)]
            S = [plsc.load_gather(gs_v, [rbase + a]) for a in range(4)]

            code = (S[0] > 0.0).astype(jnp.int32)
            code = code + 2 * (S[1] > 0.0).astype(jnp.int32)
            code = code + 4 * (S[2] > 0.0).astype(jnp.int32)
            code = code + 8 * (S[3] > 0.0).astype(jnp.int32)
            code6 = code * 6

            gt6 = (t0 + tloc) * 6
            f6 = tloc * 6
            valid = []
            for col in range(6):
                ent = plsc.load_gather(tt_v, [code6 + col])
                v_ok = ent >= 0
                valid.append(v_ok)
                fval = gt6 + jnp.where(v_ok, ent, 0)
                plsc.store_scatter(fc_v, [f6 + col], fval)
            for r in range(2):
                v_ok = valid[3 * r] & valid[3 * r + 1] & valid[3 * r + 2]
                plsc.store_scatter(vl_v, [2 * tloc + r],
                                   v_ok.astype(jnp.int32))

            m18 = tloc * 18
            for e in range(6):
                a, b = EDGE_A[e], EDGE_B[e]
                d = S[b] - S[a]
                small = jnp.abs(d) < 1e-10
                w = jnp.where(small, 0.5,
                              S[b] / jnp.where(small, 1.0, d))
                u = 1.0 - w
                base = m18 + e * 3
                plsc.store_scatter(mv_v, [base], X[a] * w + X[b] * u)
                plsc.store_scatter(mv_v, [base + 1], Y[a] * w + Y[b] * u)
                plsc.store_scatter(mv_v, [base + 2], Z[a] * w + Z[b] * u)
            return c2

        lax.fori_loop(0, G, group_body, 0)

        pltpu.sync_copy(mv_v, mv_hbm.at[pl.ds(t0 * 18, 18 * C)])
        pltpu.sync_copy(fc_v, fc_hbm.at[pl.ds(t0 * 6, 6 * C)])
        pltpu.sync_copy(vl_v, vl_hbm.at[pl.ds(t0 * 2, 2 * C)])
        return carry

    lax.fori_loop(0, CH, chunk_body, 0)


def _marching(x, y, z, s, ti_flat, tt):
    mesh = plsc.VectorSubcoreMesh(core_axis_name="c", subcore_axis_name="s")
    val_t = pltpu.VMEM((4 * C,), jnp.float32)
    fn = pl.kernel(
        _mt_body, mesh=mesh,
        out_type=[
            jax.ShapeDtypeStruct((F * 18,), jnp.float32),
            jax.ShapeDtypeStruct((F * 6,), jnp.int32),
            jax.ShapeDtypeStruct((F * 2,), jnp.int32),
        ],
        scratch_types=[
            pltpu.VMEM_SHARED((N,), jnp.float32),
            pltpu.VMEM_SHARED((N,), jnp.float32),
            pltpu.VMEM_SHARED((N,), jnp.float32),
            pltpu.VMEM_SHARED((N,), jnp.float32),
            pltpu.VMEM((4 * C,), jnp.int32),
            val_t, val_t, val_t, val_t,
            pltpu.VMEM((96,), jnp.int32),
            pltpu.VMEM((18 * C,), jnp.float32),
            pltpu.VMEM((6 * C,), jnp.int32),
            pltpu.VMEM((2 * C,), jnp.int32),
            pltpu.SemaphoreType.DMA,
        ],
        compiler_params=pltpu.CompilerParams(needs_layout_passes=False),
    )
    return fn(x, y, z, s, ti_flat, tt)


def kernel(tet_v, sdf, deform, tet_ind):
    x, y, z = _deform_verts(tet_v, deform)
    ti_flat = tet_ind.reshape(F * 4)
    tt = jnp.asarray(TRI_TABLE).reshape(96)
    mv, fc, vl = _marching(x, y, z, sdf, ti_flat, tt)
    mesh_verts = mv.reshape(F * 6, 3)
    faces = fc.reshape(F * 2, 3)
    face_valid = vl.astype(bool)
    return mesh_verts, faces, face_valid


# speedup vs baseline: 8.4501x; 6.0094x over previous
import jax
import jax.numpy as jnp
import numpy as np
from jax import lax
from jax.experimental import pallas as pl
from jax.experimental.pallas import tpu as pltpu
from jax.experimental.pallas import tpu_sc as plsc

GRID_SCALE = 0.0001

TRI_TABLE = np.array([
    [-1, -1, -1, -1, -1, -1],
    [1, 0, 2, -1, -1, -1],
    [4, 0, 3, -1, -1, -1],
    [1, 4, 2, 1, 3, 4],
    [3, 1, 5, -1, -1, -1],
    [2, 3, 0, 2, 5, 3],
    [1, 4, 0, 1, 5, 4],
    [4, 2, 5, -1, -1, -1],
    [4, 5, 2, -1, -1, -1],
    [4, 1, 0, 4, 5, 1],
    [3, 2, 0, 3, 5, 2],
    [1, 3, 5, -1, -1, -1],
    [4, 1, 2, 4, 3, 1],
    [3, 0, 4, -1, -1, -1],
    [2, 0, 1, -1, -1, -1],
    [-1, -1, -1, -1, -1, -1]], dtype=np.int32)

EDGE_A = (0, 0, 0, 1, 1, 2)
EDGE_B = (1, 2, 3, 2, 3, 3)

NC = 2
NS = 16
L = 16
NW = NC * NS

F = 300000
N = 100000

C = 640
G = C // L
CH = 15
NGRP = F // L
BASE_G = NGRP // NW
REM_G = NGRP % NW


def _pack_body(tx, ty, tz, dx, dy, dz, ox, oy, oz):
    h = GRID_SCALE / 2.0
    ox[...] = tx[...] + jnp.tanh(dx[...]) * h
    oy[...] = ty[...] + jnp.tanh(dy[...]) * h
    oz[...] = tz[...] + jnp.tanh(dz[...]) * h


def _deform_verts(tet_v, deform):
    bn = 2048
    shp = jax.ShapeDtypeStruct((N,), jnp.float32)
    spec = pl.BlockSpec((bn,), lambda i: (i,))
    return pl.pallas_call(
        _pack_body,
        grid=((N + bn - 1) // bn,),
        in_specs=[spec] * 6,
        out_specs=[spec, spec, spec],
        out_shape=[shp, shp, shp],
    )(tet_v[:, 0], tet_v[:, 1], tet_v[:, 2],
      deform[:, 0], deform[:, 1], deform[:, 2])


def _mt_body(x_hbm, y_hbm, z_hbm, s_hbm, i0_hbm, i1_hbm, i2_hbm, i3_hbm,
             tt_hbm,
             mx_hbm, my_hbm, mz_hbm, f0_hbm, f1_hbm, f2_hbm, vl_hbm,
             xs_sh, ys_sh, zs_sh, ss_sh,
             ia0, ia1, ia2, ia3,
             gx0, gx1, gx2, gx3, gy0, gy1, gy2, gy3,
             gz0, gz1, gz2, gz3, gs0, gs1, gs2, gs3,
             tt_v, mx_v, my_v, mz_v, f0_v, f1_v, f2_v, vl_v, sem):
    cid = lax.axis_index("c")
    sid = lax.axis_index("s")
    wid = sid * NC + cid
    t0w = (wid * BASE_G + jnp.minimum(wid, REM_G)) * L
    tend = t0w + (BASE_G + jnp.where(wid < REM_G, 1, 0)) * L

    @pl.when(sid == 0)
    def _():
        pltpu.sync_copy(x_hbm, xs_sh)

    @pl.when(sid == 1)
    def _():
        pltpu.sync_copy(y_hbm, ys_sh)

    @pl.when(sid == 2)
    def _():
        pltpu.sync_copy(z_hbm, zs_sh)

    @pl.when(sid == 3)
    def _():
        pltpu.sync_copy(s_hbm, ss_sh)

    pltpu.sync_copy(tt_hbm, tt_v)
    plsc.subcore_barrier()

    lane = lax.broadcasted_iota(jnp.int32, (L,), 0)
    idx_bufs = (ia0, ia1, ia2, ia3)
    ih = (i0_hbm, i1_hbm, i2_hbm, i3_hbm)
    gx = (gx0, gx1, gx2, gx3)
    gy = (gy0, gy1, gy2, gy3)
    gz = (gz0, gz1, gz2, gz3)
    gs = (gs0, gs1, gs2, gs3)

    def chunk_body(g, carry):
        t0 = jnp.minimum(t0w + g * C, tend - C)
        for a in range(4):
            pltpu.sync_copy(ih[a].at[pl.ds(t0, C)], idx_bufs[a])
        copies = []
        for a in range(4):
            copies.append(pltpu.async_copy(xs_sh.at[idx_bufs[a]], gx[a], sem))
            copies.append(pltpu.async_copy(ys_sh.at[idx_bufs[a]], gy[a], sem))
            copies.append(pltpu.async_copy(zs_sh.at[idx_bufs[a]], gz[a], sem))
            copies.append(pltpu.async_copy(ss_sh.at[idx_bufs[a]], gs[a], sem))
        for cp in copies:
            cp.wait()

        def group_body(j, c2):
            o = pl.ds(j * L, L)
            tloc = j * L + lane
            X = [gx[a][o] for a in range(4)]
            Y = [gy[a][o] for a in range(4)]
            Z = [gz[a][o] for a in range(4)]
            S = [gs[a][o] for a in range(4)]

            code = (S[0] > 0.0).astype(jnp.int32)
            code = code + 2 * (S[1] > 0.0).astype(jnp.int32)
            code = code + 4 * (S[2] > 0.0).astype(jnp.int32)
            code = code + 8 * (S[3] > 0.0).astype(jnp.int32)
            code6 = code * 6

            gt6 = (t0 + tloc) * 6
            fplane = (f0_v, f1_v, f2_v)
            valid = []
            for col in range(6):
                ent = plsc.load_gather(tt_v, [code6 + col])
                v_ok = ent >= 0
                valid.append(v_ok)
                fval = gt6 + jnp.where(v_ok, ent, 0)
                plsc.store_scatter(fplane[col % 3], [2 * tloc + col // 3],
                                   fval)
            for r in range(2):
                v_ok = valid[3 * r] & valid[3 * r + 1] & valid[3 * r + 2]
                plsc.store_scatter(vl_v, [2 * tloc + r],
                                   v_ok.astype(jnp.int32))

            m6 = tloc * 6
            for e in range(6):
                a, b = EDGE_A[e], EDGE_B[e]
                d = S[b] - S[a]
                small = jnp.abs(d) < 1e-10
                w = jnp.where(small, 0.5,
                              S[b] / jnp.where(small, 1.0, d))
                u = 1.0 - w
                plsc.store_scatter(mx_v, [m6 + e], X[a] * w + X[b] * u)
                plsc.store_scatter(my_v, [m6 + e], Y[a] * w + Y[b] * u)
                plsc.store_scatter(mz_v, [m6 + e], Z[a] * w + Z[b] * u)
            return c2

        lax.fori_loop(0, G, group_body, 0)

        pltpu.sync_copy(mx_v, mx_hbm.at[pl.ds(t0 * 6, 6 * C)])
        pltpu.sync_copy(my_v, my_hbm.at[pl.ds(t0 * 6, 6 * C)])
        pltpu.sync_copy(mz_v, mz_hbm.at[pl.ds(t0 * 6, 6 * C)])
        pltpu.sync_copy(f0_v, f0_hbm.at[pl.ds(t0 * 2, 2 * C)])
        pltpu.sync_copy(f1_v, f1_hbm.at[pl.ds(t0 * 2, 2 * C)])
        pltpu.sync_copy(f2_v, f2_hbm.at[pl.ds(t0 * 2, 2 * C)])
        pltpu.sync_copy(vl_v, vl_hbm.at[pl.ds(t0 * 2, 2 * C)])
        return carry

    lax.fori_loop(0, CH, chunk_body, 0)


def _marching(x, y, z, s, i0, i1, i2, i3, tt):
    mesh = plsc.VectorSubcoreMesh(core_axis_name="c", subcore_axis_name="s")
    tbl_t = pltpu.VMEM_SHARED((N,), jnp.float32)
    idx_t = pltpu.VMEM((C,), jnp.int32)
    val_t = pltpu.VMEM((C,), jnp.float32)
    mvo_t = jax.ShapeDtypeStruct((F * 6,), jnp.float32)
    fco_t = jax.ShapeDtypeStruct((F * 2,), jnp.int32)
    fn = pl.kernel(
        _mt_body, mesh=mesh,
        out_type=[mvo_t, mvo_t, mvo_t, fco_t, fco_t, fco_t, fco_t],
        scratch_types=[
            tbl_t, tbl_t, tbl_t, tbl_t,
            idx_t, idx_t, idx_t, idx_t,
            val_t, val_t, val_t, val_t,
            val_t, val_t, val_t, val_t,
            val_t, val_t, val_t, val_t,
            val_t, val_t, val_t, val_t,
            pltpu.VMEM((96,), jnp.int32),
            pltpu.VMEM((6 * C,), jnp.float32),
            pltpu.VMEM((6 * C,), jnp.float32),
            pltpu.VMEM((6 * C,), jnp.float32),
            pltpu.VMEM((2 * C,), jnp.int32),
            pltpu.VMEM((2 * C,), jnp.int32),
            pltpu.VMEM((2 * C,), jnp.int32),
            pltpu.VMEM((2 * C,), jnp.int32),
            pltpu.SemaphoreType.DMA,
        ],
        compiler_params=pltpu.CompilerParams(needs_layout_passes=False),
    )
    return fn(x, y, z, s, i0, i1, i2, i3, tt)


def kernel(tet_v, sdf, deform, tet_ind):
    x, y, z = _deform_verts(tet_v, deform)
    tt = jnp.asarray(TRI_TABLE).reshape(96)
    mx, my, mz, f0, f1, f2, vl = _marching(
        x, y, z, sdf,
        tet_ind[:, 0], tet_ind[:, 1], tet_ind[:, 2], tet_ind[:, 3], tt)
    mesh_verts = jnp.stack([mx, my, mz], axis=1)
    faces = jnp.stack([f0, f1, f2], axis=1)
    face_valid = vl.astype(bool)
    return mesh_verts, faces, face_valid


# speedup vs baseline: 10.1869x; 1.2055x over previous
import jax
import jax.numpy as jnp
import numpy as np
from jax import lax
from jax.experimental import pallas as pl
from jax.experimental.pallas import tpu as pltpu
from jax.experimental.pallas import tpu_sc as plsc

GRID_SCALE = 0.0001

TRI_TABLE = np.array([
    [-1, -1, -1, -1, -1, -1],
    [1, 0, 2, -1, -1, -1],
    [4, 0, 3, -1, -1, -1],
    [1, 4, 2, 1, 3, 4],
    [3, 1, 5, -1, -1, -1],
    [2, 3, 0, 2, 5, 3],
    [1, 4, 0, 1, 5, 4],
    [4, 2, 5, -1, -1, -1],
    [4, 5, 2, -1, -1, -1],
    [4, 1, 0, 4, 5, 1],
    [3, 2, 0, 3, 5, 2],
    [1, 3, 5, -1, -1, -1],
    [4, 1, 2, 4, 3, 1],
    [3, 0, 4, -1, -1, -1],
    [2, 0, 1, -1, -1, -1],
    [-1, -1, -1, -1, -1, -1]], dtype=np.int32)

EDGE_A = (0, 0, 0, 1, 1, 2)
EDGE_B = (1, 2, 3, 2, 3, 3)

NC = 2
NS = 16
L = 16
NW = NC * NS

F = 300000
N = 100000

C = 1184
G = C // L
CH = 8
NGRP = F // L
BASE_G = NGRP // NW
REM_G = NGRP % NW


def _pack_body(tx, ty, tz, dx, dy, dz, ox, oy, oz):
    h = GRID_SCALE / 2.0
    ox[...] = tx[...] + jnp.tanh(dx[...]) * h
    oy[...] = ty[...] + jnp.tanh(dy[...]) * h
    oz[...] = tz[...] + jnp.tanh(dz[...]) * h


def _deform_verts(tet_v, deform):
    bn = 2048
    shp = jax.ShapeDtypeStruct((N,), jnp.float32)
    spec = pl.BlockSpec((bn,), lambda i: (i,))
    return pl.pallas_call(
        _pack_body,
        grid=((N + bn - 1) // bn,),
        in_specs=[spec] * 6,
        out_specs=[spec, spec, spec],
        out_shape=[shp, shp, shp],
    )(tet_v[:, 0], tet_v[:, 1], tet_v[:, 2],
      deform[:, 0], deform[:, 1], deform[:, 2])


def _mt_body(x_hbm, y_hbm, z_hbm, s_hbm, i0_hbm, i1_hbm, i2_hbm, i3_hbm,
             tt_hbm,
             mx_hbm, my_hbm, mz_hbm, f0_hbm, f1_hbm, f2_hbm, vl_hbm,
             xs_sh, ys_sh, zs_sh, ss_sh,
             ia0, ia1, ia2, ia3, ib0, ib1, ib2, ib3,
             gax0, gax1, gax2, gax3, gay0, gay1, gay2, gay3,
             gaz0, gaz1, gaz2, gaz3, gas0, gas1, gas2, gas3,
             gbx0, gbx1, gbx2, gbx3, gby0, gby1, gby2, gby3,
             gbz0, gbz1, gbz2, gbz3, gbs0, gbs1, gbs2, gbs3,
             tt_v, mx_v, my_v, mz_v, f0_v, f1_v, f2_v, vl_v,
             sem_a, sem_b):
    cid = lax.axis_index("c")
    sid = lax.axis_index("s")
    wid = sid * NC + cid
    t0w = (wid * BASE_G + jnp.minimum(wid, REM_G)) * L
    tend = t0w + (BASE_G + jnp.where(wid < REM_G, 1, 0)) * L

    @pl.when(sid == 0)
    def _():
        pltpu.sync_copy(x_hbm, xs_sh)

    @pl.when(sid == 1)
    def _():
        pltpu.sync_copy(y_hbm, ys_sh)

    @pl.when(sid == 2)
    def _():
        pltpu.sync_copy(z_hbm, zs_sh)

    @pl.when(sid == 3)
    def _():
        pltpu.sync_copy(s_hbm, ss_sh)

    pltpu.sync_copy(tt_hbm, tt_v)
    plsc.subcore_barrier()

    lane = lax.broadcasted_iota(jnp.int32, (L,), 0)
    ih = (i0_hbm, i1_hbm, i2_hbm, i3_hbm)
    tabs = (xs_sh, ys_sh, zs_sh, ss_sh)
    bufs = {
        0: ((ia0, ia1, ia2, ia3),
            ((gax0, gax1, gax2, gax3), (gay0, gay1, gay2, gay3),
             (gaz0, gaz1, gaz2, gaz3), (gas0, gas1, gas2, gas3)),
            sem_a),
        1: ((ib0, ib1, ib2, ib3),
            ((gbx0, gbx1, gbx2, gbx3), (gby0, gby1, gby2, gby3),
             (gbz0, gbz1, gbz2, gbz3), (gbs0, gbs1, gbs2, gbs3)),
            sem_b),
    }

    def fire(t0, p):
        idx, g, sem = bufs[p]
        for a in range(4):
            pltpu.sync_copy(ih[a].at[pl.ds(t0, C)], idx[a])
        for a in range(4):
            for comp in range(4):
                pltpu.async_copy(tabs[comp].at[idx[a]], g[comp][a], sem)

    def drain(p):
        idx, g, sem = bufs[p]
        for a in range(4):
            for comp in range(4):
                pltpu.make_async_copy(tabs[comp].at[idx[a]], g[comp][a],
                                      sem).wait()

    def compute(t0, p):
        _, g, _ = bufs[p]
        gx, gy, gz, gs = g

        def group_body(j, c2):
            o = pl.ds(j * L, L)
            tloc = j * L + lane
            X = [gx[a][o] for a in range(4)]
            Y = [gy[a][o] for a in range(4)]
            Z = [gz[a][o] for a in range(4)]
            S = [gs[a][o] for a in range(4)]

            code = (S[0] > 0.0).astype(jnp.int32)
            code = code + 2 * (S[1] > 0.0).astype(jnp.int32)
            code = code + 4 * (S[2] > 0.0).astype(jnp.int32)
            code = code + 8 * (S[3] > 0.0).astype(jnp.int32)
            code6 = code * 6

            gt6 = (t0 + tloc) * 6
            fplane = (f0_v, f1_v, f2_v)
            valid = []
            for col in range(6):
                ent = plsc.load_gather(tt_v, [code6 + col])
                v_ok = ent >= 0
                valid.append(v_ok)
                fval = gt6 + jnp.where(v_ok, ent, 0)
                plsc.store_scatter(fplane[col % 3], [2 * tloc + col // 3],
                                   fval)
            for r in range(2):
                v_ok = valid[3 * r] & valid[3 * r + 1] & valid[3 * r + 2]
                plsc.store_scatter(vl_v, [2 * tloc + r],
                                   v_ok.astype(jnp.int32))

            m6 = tloc * 6
            for e in range(6):
                a, b = EDGE_A[e], EDGE_B[e]
                d = S[b] - S[a]
                small = jnp.abs(d) < 1e-10
                w = jnp.where(small, 0.5,
                              S[b] / jnp.where(small, 1.0, d))
                u = 1.0 - w
                plsc.store_scatter(mx_v, [m6 + e], X[a] * w + X[b] * u)
                plsc.store_scatter(my_v, [m6 + e], Y[a] * w + Y[b] * u)
                plsc.store_scatter(mz_v, [m6 + e], Z[a] * w + Z[b] * u)
            return c2

        lax.fori_loop(0, G, group_body, 0)

        pltpu.sync_copy(mx_v, mx_hbm.at[pl.ds(t0 * 6, 6 * C)])
        pltpu.sync_copy(my_v, my_hbm.at[pl.ds(t0 * 6, 6 * C)])
        pltpu.sync_copy(mz_v, mz_hbm.at[pl.ds(t0 * 6, 6 * C)])
        pltpu.sync_copy(f0_v, f0_hbm.at[pl.ds(t0 * 2, 2 * C)])
        pltpu.sync_copy(f1_v, f1_hbm.at[pl.ds(t0 * 2, 2 * C)])
        pltpu.sync_copy(f2_v, f2_hbm.at[pl.ds(t0 * 2, 2 * C)])
        pltpu.sync_copy(vl_v, vl_hbm.at[pl.ds(t0 * 2, 2 * C)])

    def tet0(g):
        return jnp.minimum(t0w + g * C, tend - C)

    fire(tet0(0), 0)

    def pair_body(k, carry):
        fire(tet0(2 * k + 1), 1)
        drain(0)
        compute(tet0(2 * k), 0)

        @pl.when(k < CH // 2 - 1)
        def _():
            fire(tet0(2 * k + 2), 0)

        drain(1)
        compute(tet0(2 * k + 1), 1)
        return carry

    lax.fori_loop(0, CH // 2, pair_body, 0)


def _marching(x, y, z, s, i0, i1, i2, i3, tt):
    mesh = plsc.VectorSubcoreMesh(core_axis_name="c", subcore_axis_name="s")
    tbl_t = pltpu.VMEM_SHARED((N,), jnp.float32)
    idx_t = pltpu.VMEM((C,), jnp.int32)
    val_t = pltpu.VMEM((C,), jnp.float32)
    mvo_t = jax.ShapeDtypeStruct((F * 6,), jnp.float32)
    fco_t = jax.ShapeDtypeStruct((F * 2,), jnp.int32)
    fn = pl.kernel(
        _mt_body, mesh=mesh,
        out_type=[mvo_t, mvo_t, mvo_t, fco_t, fco_t, fco_t, fco_t],
        scratch_types=[
            tbl_t, tbl_t, tbl_t, tbl_t,
        ] + [idx_t] * 8 + [val_t] * 32 + [
            pltpu.VMEM((96,), jnp.int32),
            pltpu.VMEM((6 * C,), jnp.float32),
            pltpu.VMEM((6 * C,), jnp.float32),
            pltpu.VMEM((6 * C,), jnp.float32),
            pltpu.VMEM((2 * C,), jnp.int32),
            pltpu.VMEM((2 * C,), jnp.int32),
            pltpu.VMEM((2 * C,), jnp.int32),
            pltpu.VMEM((2 * C,), jnp.int32),
            pltpu.SemaphoreType.DMA,
            pltpu.SemaphoreType.DMA,
        ],
        compiler_params=pltpu.CompilerParams(needs_layout_passes=False),
    )
    return fn(x, y, z, s, i0, i1, i2, i3, tt)


def kernel(tet_v, sdf, deform, tet_ind):
    x, y, z = _deform_verts(tet_v, deform)
    tt = jnp.asarray(TRI_TABLE).reshape(96)
    mx, my, mz, f0, f1, f2, vl = _marching(
        x, y, z, sdf,
        tet_ind[:, 0], tet_ind[:, 1], tet_ind[:, 2], tet_ind[:, 3], tt)
    mesh_verts = jnp.stack([mx, my, mz], axis=1)
    faces = jnp.stack([f0, f1, f2], axis=1)
    face_valid = vl.astype(bool)
    return mesh_verts, faces, face_valid
